# baseline (speedup 1.0000x reference)
import jax
import jax.numpy as jnp
from jax import lax
from jax.experimental import pallas as pl
from jax.experimental.pallas import tpu as pltpu
from jax.experimental.pallas import tpu_sc as plsc

_B, _N = 64, 8192
_K = _N // 2
_NC, _NS = 2, 16
_NW = _NC * _NS
_RPW = _B // _NW
_L = 16
_VPR = _N // _L
_NBINS = 256
_BIG = 2 ** 30


def _scan_select(hist_ref, kk):
    carry = jnp.int32(0)
    cnt_vec = jnp.zeros((_L,), jnp.int32)
    a_vec = jnp.zeros((_L,), jnp.int32)
    m_vec = jnp.full((_L,), _BIG, jnp.int32)
    for j in range(_NBINS // _L):
        v = hist_ref[pl.ds((_NBINS // _L - 1 - j) * _L, _L)]
        rv = lax.rev(v, (0,))
        cs = plsc.cumsum(rv) + carry
        ge = cs >= kk
        cnt_vec = cnt_vec + ge.astype(jnp.int32)
        a_vec = jnp.maximum(a_vec, jnp.where(ge, 0, cs))
        m_vec = jnp.minimum(m_vec, jnp.where(ge, cs, _BIG))
        carry = jnp.max(cs)
    b = jnp.sum(cnt_vec) - 1
    return b, jnp.max(a_vec), jnp.min(m_vec)


def _zero_hist(hist_ref):
    zeros = jnp.zeros((_L,), jnp.int32)
    for i in range(_NBINS // _L):
        hist_ref[pl.ds(i * _L, _L)] = zeros


def _process_row(xrow, krow, hist_ref):
    ones = jnp.ones((_L,), jnp.int32)

    _zero_hist(hist_ref)

    def body1(i, c):
        xvec = xrow[pl.ds(i * _L, _L)]
        u = lax.bitcast_convert_type(xvec, jnp.uint32)
        ku = jnp.where(u >= jnp.uint32(0x80000000), ~u, u | jnp.uint32(0x80000000))
        krow[pl.ds(i * _L, _L)] = ku
        digit = jnp.right_shift(ku, jnp.uint32(24)).astype(jnp.int32)
        plsc.addupdate_scatter(hist_ref, [digit], ones)
        return c

    lax.fori_loop(0, _VPR, body1, 0)

    kk = jnp.int32(_K)
    b, A, M = _scan_select(hist_ref, kk)
    kk = kk - A
    prefix = b.astype(jnp.uint32)

    for shift in (16, 8, 0):
        _zero_hist(hist_ref)

        def bodyp(i, c, shift=shift, prefix=prefix):
            kvec = krow[pl.ds(i * _L, _L)]
            cand = jnp.right_shift(kvec, jnp.uint32(shift + 8)) == prefix
            digit = (jnp.right_shift(kvec, jnp.uint32(shift))
                     & jnp.uint32(0xFF)).astype(jnp.int32)
            plsc.addupdate_scatter(hist_ref, [digit], ones, mask=cand)
            return c

        lax.fori_loop(0, _VPR, bodyp, 0)
        b, A, M = _scan_select(hist_ref, kk)
        kk = kk - A
        prefix = jnp.left_shift(prefix, jnp.uint32(8)) | b.astype(jnp.uint32)

    t = prefix
    rem = (M - A) - kk

    def body_mask(i, c):
        kvec = krow[pl.ds(i * _L, _L)]
        xvec = xrow[pl.ds(i * _L, _L)]
        xrow[pl.ds(i * _L, _L)] = jnp.where(kvec >= t, xvec, jnp.float32(0.0))
        return c

    lax.fori_loop(0, _VPR, body_mask, 0)

    @pl.when(rem > 0)
    def _fixup():
        def bodyf(j, r):
            i = _VPR - 1 - j
            kvec = krow[pl.ds(i * _L, _L)]
            eq = kvec == t
            rcs = plsc.cumsum(lax.rev(eq.astype(jnp.int32), (0,)))
            zmask = eq & (lax.rev(rcs, (0,)) <= r)
            xvec = xrow[pl.ds(i * _L, _L)]
            xrow[pl.ds(i * _L, _L)] = jnp.where(zmask, jnp.float32(0.0), xvec)
            return r - jnp.max(rcs)

        lax.fori_loop(0, _VPR, bodyf, rem)


def _sc_body(x_hbm, o_hbm, xv, kv, hist):
    wid = lax.axis_index("s") * _NC + lax.axis_index("c")
    base = wid * (_RPW * _N)
    pltpu.sync_copy(x_hbm.at[pl.ds(base, _RPW * _N)], xv)
    for r in range(_RPW):
        _process_row(xv.at[pl.ds(r * _N, _N)], kv, hist)
    pltpu.sync_copy(xv, o_hbm.at[pl.ds(base, _RPW * _N)])


@jax.jit
def kernel(input):
    mesh = plsc.VectorSubcoreMesh(
        core_axis_name="c", subcore_axis_name="s",
        num_cores=_NC, num_subcores=_NS,
    )
    f = pl.kernel(
        _sc_body,
        out_type=jax.ShapeDtypeStruct((_B * _N,), jnp.float32),
        mesh=mesh,
        compiler_params=pltpu.CompilerParams(needs_layout_passes=False),
        scratch_types=[
            pltpu.VMEM((_RPW * _N,), jnp.float32),
            pltpu.VMEM((_N,), jnp.uint32),
            pltpu.VMEM((_NBINS,), jnp.int32),
        ],
    )
    return f(input.reshape(-1)).reshape(_B, _N)


# speedup vs baseline: 1.7966x; 1.7966x over previous
import jax
import jax.numpy as jnp
from jax import lax
from jax.experimental import pallas as pl
from jax.experimental.pallas import tpu as pltpu
from jax.experimental.pallas import tpu_sc as plsc

_B, _N = 64, 8192
_K = _N // 2
_NC, _NS = 2, 16
_NW = _NC * _NS
_RPW = _B // _NW
_L = 16
_VPR = _N // _L
_NBINS = 256
_BIG = 2 ** 30


def _scan_select(hist_ref, kk):
    carry = jnp.int32(0)
    cnt_vec = jnp.zeros((_L,), jnp.int32)
    a_vec = jnp.zeros((_L,), jnp.int32)
    m_vec = jnp.full((_L,), _BIG, jnp.int32)
    for j in range(_NBINS // _L):
        v = hist_ref[pl.ds((_NBINS // _L - 1 - j) * _L, _L)]
        rv = lax.rev(v, (0,))
        cs = plsc.cumsum(rv) + carry
        ge = cs >= kk
        cnt_vec = cnt_vec + ge.astype(jnp.int32)
        a_vec = jnp.maximum(a_vec, jnp.where(ge, 0, cs))
        m_vec = jnp.minimum(m_vec, jnp.where(ge, cs, _BIG))
        carry = jnp.max(cs)
    b = jnp.sum(cnt_vec) - 1
    return b, jnp.max(a_vec), jnp.min(m_vec)


def _zero_hist(hist_ref):
    zeros = jnp.zeros((_L,), jnp.int32)
    for i in range(_NBINS // _L):
        hist_ref[pl.ds(i * _L, _L)] = zeros


def _process_row(xrow, krow, hist_ref):
    ones = jnp.ones((_L,), jnp.int32)

    _zero_hist(hist_ref)

    @plsc.parallel_loop(0, _N, step=_L, unroll=8)
    def body1(i):
        xvec = xrow[pl.ds(i, _L)]
        u = lax.bitcast_convert_type(xvec, jnp.uint32)
        ku = jnp.where(u >= jnp.uint32(0x80000000), ~u, u | jnp.uint32(0x80000000))
        krow[pl.ds(i, _L)] = ku
        digit = jnp.right_shift(ku, jnp.uint32(24)).astype(jnp.int32)
        plsc.addupdate_scatter(hist_ref, [digit], ones)

    kk = jnp.int32(_K)
    b, A, M = _scan_select(hist_ref, kk)
    kk = kk - A
    prefix = b.astype(jnp.uint32)

    for shift in (16, 8, 0):
        _zero_hist(hist_ref)

        @plsc.parallel_loop(0, _N, step=_L, unroll=8)
        def bodyp(i, shift=shift, prefix=prefix):
            kvec = krow[pl.ds(i, _L)]
            cand = jnp.right_shift(kvec, jnp.uint32(shift + 8)) == prefix
            digit = (jnp.right_shift(kvec, jnp.uint32(shift))
                     & jnp.uint32(0xFF)).astype(jnp.int32)
            plsc.addupdate_scatter(hist_ref, [digit], ones, mask=cand)
        b, A, M = _scan_select(hist_ref, kk)
        kk = kk - A
        prefix = jnp.left_shift(prefix, jnp.uint32(8)) | b.astype(jnp.uint32)

    t = prefix
    rem = (M - A) - kk

    @plsc.parallel_loop(0, _N, step=_L, unroll=8)
    def body_mask(i):
        kvec = krow[pl.ds(i, _L)]
        xvec = xrow[pl.ds(i, _L)]
        xrow[pl.ds(i, _L)] = jnp.where(kvec >= t, xvec, jnp.float32(0.0))

    @pl.when(rem > 0)
    def _fixup():
        def bodyf(j, r):
            i = _VPR - 1 - j
            kvec = krow[pl.ds(i * _L, _L)]
            eq = kvec == t
            rcs = plsc.cumsum(lax.rev(eq.astype(jnp.int32), (0,)))
            zmask = eq & (lax.rev(rcs, (0,)) <= r)
            xvec = xrow[pl.ds(i * _L, _L)]
            xrow[pl.ds(i * _L, _L)] = jnp.where(zmask, jnp.float32(0.0), xvec)
            return r - jnp.max(rcs)

        lax.fori_loop(0, _VPR, bodyf, rem)


def _sc_body(x_hbm, o_hbm, xv, kv, hist):
    wid = lax.axis_index("s") * _NC + lax.axis_index("c")
    base = wid * (_RPW * _N)
    pltpu.sync_copy(x_hbm.at[pl.ds(base, _RPW * _N)], xv)
    for r in range(_RPW):
        _process_row(xv.at[pl.ds(r * _N, _N)], kv, hist)
    pltpu.sync_copy(xv, o_hbm.at[pl.ds(base, _RPW * _N)])


@jax.jit
def kernel(input):
    mesh = plsc.VectorSubcoreMesh(
        core_axis_name="c", subcore_axis_name="s",
        num_cores=_NC, num_subcores=_NS,
    )
    f = pl.kernel(
        _sc_body,
        out_type=jax.ShapeDtypeStruct((_B * _N,), jnp.float32),
        mesh=mesh,
        compiler_params=pltpu.CompilerParams(needs_layout_passes=False),
        scratch_types=[
            pltpu.VMEM((_RPW * _N,), jnp.float32),
            pltpu.VMEM((_N,), jnp.uint32),
            pltpu.VMEM((_NBINS,), jnp.int32),
        ],
    )
    return f(input.reshape(-1)).reshape(_B, _N)


# speedup vs baseline: 1.8475x; 1.0283x over previous
import jax
import jax.numpy as jnp
from jax import lax
from jax.experimental import pallas as pl
from jax.experimental.pallas import tpu as pltpu
from jax.experimental.pallas import tpu_sc as plsc

_B, _N = 64, 8192
_K = _N // 2
_NC, _NS = 2, 16
_NW = _NC * _NS
_RPW = _B // _NW
_L = 16
_VPR = _N // _L
_NBINS = 256
_BIG = 2 ** 30


def _scan_select(hist_ref, kk):
    carry = jnp.int32(0)
    cnt_vec = jnp.zeros((_L,), jnp.int32)
    a_vec = jnp.zeros((_L,), jnp.int32)
    m_vec = jnp.full((_L,), _BIG, jnp.int32)
    for j in range(_NBINS // _L):
        v = hist_ref[pl.ds((_NBINS // _L - 1 - j) * _L, _L)]
        rv = lax.rev(v, (0,))
        cs = plsc.cumsum(rv) + carry
        ge = cs >= kk
        cnt_vec = cnt_vec + ge.astype(jnp.int32)
        a_vec = jnp.maximum(a_vec, jnp.where(ge, 0, cs))
        m_vec = jnp.minimum(m_vec, jnp.where(ge, cs, _BIG))
        carry = jnp.max(cs)
    b = jnp.sum(cnt_vec) - 1
    return b, jnp.max(a_vec), jnp.min(m_vec)


def _zero_hist(hist_ref):
    zeros = jnp.zeros((_L,), jnp.int32)
    for i in range(_NBINS // _L):
        hist_ref[pl.ds(i * _L, _L)] = zeros


def _process_row(xrow, krow, hist_ref, cb1, cb2):
    ones = jnp.ones((_L,), jnp.int32)

    _zero_hist(hist_ref)

    @plsc.parallel_loop(0, _N, step=_L, unroll=8)
    def body1(i):
        xvec = xrow[pl.ds(i, _L)]
        u = lax.bitcast_convert_type(xvec, jnp.uint32)
        ku = jnp.where(u >= jnp.uint32(0x80000000), ~u, u | jnp.uint32(0x80000000))
        krow[pl.ds(i, _L)] = ku
        digit = jnp.right_shift(ku, jnp.uint32(24)).astype(jnp.int32)
        plsc.addupdate_scatter(hist_ref, [digit], ones)

    kk = jnp.int32(_K)
    b, A, M = _scan_select(hist_ref, kk)
    kk = kk - A
    prefix = b.astype(jnp.uint32)

    _zero_hist(hist_ref)

    @plsc.parallel_loop(0, _N, step=_L, unroll=8,
                        carry=jnp.zeros((_L,), jnp.int32))
    def body2(i, off_vec):
        kvec = krow[pl.ds(i, _L)]
        cand = jnp.right_shift(kvec, jnp.uint32(24)) == prefix
        digit = (jnp.right_shift(kvec, jnp.uint32(16))
                 & jnp.uint32(0xFF)).astype(jnp.int32)
        plsc.addupdate_scatter(hist_ref, [digit], ones, mask=cand)
        rank = plsc.cumsum(cand.astype(jnp.int32))
        idx = off_vec + rank - 1
        plsc.store_scatter(cb1, [idx], lax.bitcast_convert_type(kvec, jnp.int32),
                           mask=cand)
        return off_vec + plsc.all_reduce_population_count(cand)

    ncand1 = jnp.max(body2)
    b, A, M = _scan_select(hist_ref, kk)
    kk = kk - A
    prefix = jnp.left_shift(prefix, jnp.uint32(8)) | b.astype(jnp.uint32)

    iota = lax.broadcasted_iota(jnp.int32, (_L,), 0)
    _zero_hist(hist_ref)

    def body3(i, off_vec):
        base = i * _L
        kvec = lax.bitcast_convert_type(cb1[pl.ds(base, _L)], jnp.uint32)
        valid = (iota + base) < ncand1
        cand = valid & (jnp.right_shift(kvec, jnp.uint32(16)) == prefix)
        digit = (jnp.right_shift(kvec, jnp.uint32(8))
                 & jnp.uint32(0xFF)).astype(jnp.int32)
        plsc.addupdate_scatter(hist_ref, [digit], ones, mask=cand)
        rank = plsc.cumsum(cand.astype(jnp.int32))
        idx = off_vec + rank - 1
        plsc.store_scatter(cb2, [idx], lax.bitcast_convert_type(kvec, jnp.int32),
                           mask=cand)
        return off_vec + plsc.all_reduce_population_count(cand)

    nvec3 = (ncand1 + (_L - 1)) // _L
    ncand2 = jnp.max(lax.fori_loop(0, nvec3, body3,
                                   jnp.zeros((_L,), jnp.int32)))
    b, A, M = _scan_select(hist_ref, kk)
    kk = kk - A
    prefix = jnp.left_shift(prefix, jnp.uint32(8)) | b.astype(jnp.uint32)

    _zero_hist(hist_ref)

    def body4(i, c):
        base = i * _L
        kvec = lax.bitcast_convert_type(cb2[pl.ds(base, _L)], jnp.uint32)
        cand = (((iota + base) < ncand2)
                & (jnp.right_shift(kvec, jnp.uint32(8)) == prefix))
        digit = (kvec & jnp.uint32(0xFF)).astype(jnp.int32)
        plsc.addupdate_scatter(hist_ref, [digit], ones, mask=cand)
        return c

    nvec4 = (ncand2 + (_L - 1)) // _L
    lax.fori_loop(0, nvec4, body4, 0)
    b, A, M = _scan_select(hist_ref, kk)
    kk = kk - A
    prefix = jnp.left_shift(prefix, jnp.uint32(8)) | b.astype(jnp.uint32)

    t = prefix
    rem = (M - A) - kk

    @plsc.parallel_loop(0, _N, step=_L, unroll=8)
    def body_mask(i):
        kvec = krow[pl.ds(i, _L)]
        xvec = xrow[pl.ds(i, _L)]
        xrow[pl.ds(i, _L)] = jnp.where(kvec >= t, xvec, jnp.float32(0.0))

    @pl.when(rem > 0)
    def _fixup():
        def bodyf(j, r):
            i = _VPR - 1 - j
            kvec = krow[pl.ds(i * _L, _L)]
            eq = kvec == t
            rcs = plsc.cumsum(lax.rev(eq.astype(jnp.int32), (0,)))
            zmask = eq & (lax.rev(rcs, (0,)) <= r)
            xvec = xrow[pl.ds(i * _L, _L)]
            xrow[pl.ds(i * _L, _L)] = jnp.where(zmask, jnp.float32(0.0), xvec)
            return r - jnp.max(rcs)

        lax.fori_loop(0, _VPR, bodyf, rem)


def _sc_body(x_hbm, o_hbm, xv, kv, hist, cb1, cb2):
    wid = lax.axis_index("s") * _NC + lax.axis_index("c")
    base = wid * (_RPW * _N)
    pltpu.sync_copy(x_hbm.at[pl.ds(base, _RPW * _N)], xv)
    for r in range(_RPW):
        _process_row(xv.at[pl.ds(r * _N, _N)], kv, hist, cb1, cb2)
    pltpu.sync_copy(xv, o_hbm.at[pl.ds(base, _RPW * _N)])


@jax.jit
def kernel(input):
    mesh = plsc.VectorSubcoreMesh(
        core_axis_name="c", subcore_axis_name="s",
        num_cores=_NC, num_subcores=_NS,
    )
    f = pl.kernel(
        _sc_body,
        out_type=jax.ShapeDtypeStruct((_B * _N,), jnp.float32),
        mesh=mesh,
        compiler_params=pltpu.CompilerParams(needs_layout_passes=False),
        scratch_types=[
            pltpu.VMEM((_RPW * _N,), jnp.float32),
            pltpu.VMEM((_N,), jnp.uint32),
            pltpu.VMEM((_NBINS,), jnp.int32),
            pltpu.VMEM((_N,), jnp.int32),
            pltpu.VMEM((_N,), jnp.int32),
        ],
    )
    return f(input.reshape(-1)).reshape(_B, _N)


# speedup vs baseline: 1.9678x; 1.0651x over previous
import jax
import jax.numpy as jnp
from jax import lax
from jax.experimental import pallas as pl
from jax.experimental.pallas import tpu as pltpu
from jax.experimental.pallas import tpu_sc as plsc

_B, _N = 64, 8192
_K = _N // 2
_NC, _NS = 2, 16
_NW = _NC * _NS
_RPW = _B // _NW
_L = 16
_VPR = _N // _L
_NBINS = 256
_BIG = 2 ** 30


def _scan_select(hist_ref, kk):
    carry = jnp.int32(0)
    cnt_vec = jnp.zeros((_L,), jnp.int32)
    a_vec = jnp.zeros((_L,), jnp.int32)
    m_vec = jnp.full((_L,), _BIG, jnp.int32)
    for j in range(_NBINS // _L):
        v = hist_ref[pl.ds((_NBINS // _L - 1 - j) * _L, _L)]
        rv = lax.rev(v, (0,))
        cs = plsc.cumsum(rv) + carry
        ge = cs >= kk
        cnt_vec = cnt_vec + ge.astype(jnp.int32)
        a_vec = jnp.maximum(a_vec, jnp.where(ge, 0, cs))
        m_vec = jnp.minimum(m_vec, jnp.where(ge, cs, _BIG))
        carry = jnp.max(cs)
    b = jnp.sum(cnt_vec) - 1
    return b, jnp.max(a_vec), jnp.min(m_vec)


def _zero_hist(hist_ref):
    zeros = jnp.zeros((_L,), jnp.int32)
    for i in range(_NBINS // _L):
        hist_ref[pl.ds(i * _L, _L)] = zeros


def _process_row(xrow, krow, hist_ref, cb1, cb2):
    ones = jnp.ones((_L,), jnp.int32)

    _zero_hist(hist_ref)

    @plsc.parallel_loop(0, _N, step=_L, unroll=8)
    def body1(i):
        xvec = xrow[pl.ds(i, _L)]
        u = lax.bitcast_convert_type(xvec, jnp.uint32)
        ku = jnp.where(u >= jnp.uint32(0x80000000), ~u, u | jnp.uint32(0x80000000))
        krow[pl.ds(i, _L)] = ku
        digit = jnp.right_shift(ku, jnp.uint32(24)).astype(jnp.int32)
        plsc.addupdate_scatter(hist_ref, [digit], ones)

    kk = jnp.int32(_K)
    b, A, M = _scan_select(hist_ref, kk)
    kk = kk - A
    prefix = b.astype(jnp.uint32)

    _zero_hist(hist_ref)

    @plsc.parallel_loop(0, _N, step=_L, unroll=8,
                        carry=jnp.zeros((_L,), jnp.int32))
    def body2(i, off_vec):
        kvec = krow[pl.ds(i, _L)]
        cand = jnp.right_shift(kvec, jnp.uint32(24)) == prefix
        digit = (jnp.right_shift(kvec, jnp.uint32(16))
                 & jnp.uint32(0xFF)).astype(jnp.int32)
        plsc.addupdate_scatter(hist_ref, [digit], ones, mask=cand)
        rank = plsc.cumsum(cand.astype(jnp.int32))
        idx = off_vec + rank - 1
        plsc.store_scatter(cb1, [idx], lax.bitcast_convert_type(kvec, jnp.int32),
                           mask=cand)
        return off_vec + plsc.all_reduce_population_count(cand)

    ncand1 = jnp.max(body2)
    b, A, M = _scan_select(hist_ref, kk)
    kk = kk - A
    prefix = jnp.left_shift(prefix, jnp.uint32(8)) | b.astype(jnp.uint32)

    iota = lax.broadcasted_iota(jnp.int32, (_L,), 0)
    _zero_hist(hist_ref)

    def body3(i, off_vec):
        base = i * _L
        kvec = lax.bitcast_convert_type(cb1[pl.ds(base, _L)], jnp.uint32)
        valid = (iota + base) < ncand1
        cand = valid & (jnp.right_shift(kvec, jnp.uint32(16)) == prefix)
        digit = (jnp.right_shift(kvec, jnp.uint32(8))
                 & jnp.uint32(0xFF)).astype(jnp.int32)
        plsc.addupdate_scatter(hist_ref, [digit], ones, mask=cand)
        rank = plsc.cumsum(cand.astype(jnp.int32))
        idx = off_vec + rank - 1
        plsc.store_scatter(cb2, [idx], lax.bitcast_convert_type(kvec, jnp.int32),
                           mask=cand)
        return off_vec + plsc.all_reduce_population_count(cand)

    nvec3 = (ncand1 + (_L - 1)) // _L
    ncand2 = jnp.max(lax.fori_loop(0, nvec3, body3,
                                   jnp.zeros((_L,), jnp.int32)))
    b, A, M = _scan_select(hist_ref, kk)
    kk = kk - A
    prefix = jnp.left_shift(prefix, jnp.uint32(8)) | b.astype(jnp.uint32)

    _zero_hist(hist_ref)

    def body4(i, c):
        base = i * _L
        kvec = lax.bitcast_convert_type(cb2[pl.ds(base, _L)], jnp.uint32)
        cand = (((iota + base) < ncand2)
                & (jnp.right_shift(kvec, jnp.uint32(8)) == prefix))
        digit = (kvec & jnp.uint32(0xFF)).astype(jnp.int32)
        plsc.addupdate_scatter(hist_ref, [digit], ones, mask=cand)
        return c

    nvec4 = (ncand2 + (_L - 1)) // _L
    lax.fori_loop(0, nvec4, body4, 0)
    b, A, M = _scan_select(hist_ref, kk)
    kk = kk - A
    prefix = jnp.left_shift(prefix, jnp.uint32(8)) | b.astype(jnp.uint32)

    t = prefix
    rem = (M - A) - kk

    @plsc.parallel_loop(0, _N, step=_L, unroll=8)
    def body_mask(i):
        kvec = krow[pl.ds(i, _L)]
        xvec = xrow[pl.ds(i, _L)]
        xrow[pl.ds(i, _L)] = jnp.where(kvec >= t, xvec, jnp.float32(0.0))

    @pl.when(rem > 0)
    def _fixup():
        def bodyf(j, r):
            i = _VPR - 1 - j
            kvec = krow[pl.ds(i * _L, _L)]
            eq = kvec == t
            rcs = plsc.cumsum(lax.rev(eq.astype(jnp.int32), (0,)))
            zmask = eq & (lax.rev(rcs, (0,)) <= r)
            xvec = xrow[pl.ds(i * _L, _L)]
            xrow[pl.ds(i * _L, _L)] = jnp.where(zmask, jnp.float32(0.0), xvec)
            return r - jnp.max(rcs)

        lax.fori_loop(0, _VPR, bodyf, rem)


def _sc_body(x_hbm, o_hbm, xv, kv, hist, cb1, cb2):
    wid = lax.axis_index("s") * _NC + lax.axis_index("c")
    base = wid * _RPW
    for r in range(_RPW):
        pltpu.sync_copy(x_hbm.at[base + r], xv.at[pl.ds(r * _N, _N)])
    for r in range(_RPW):
        _process_row(xv.at[pl.ds(r * _N, _N)], kv, hist, cb1, cb2)
    for r in range(_RPW):
        pltpu.sync_copy(xv.at[pl.ds(r * _N, _N)], o_hbm.at[base + r])


@jax.jit
def kernel(input):
    mesh = plsc.VectorSubcoreMesh(
        core_axis_name="c", subcore_axis_name="s",
        num_cores=_NC, num_subcores=_NS,
    )
    f = pl.kernel(
        _sc_body,
        out_type=jax.ShapeDtypeStruct((_B, _N), jnp.float32),
        mesh=mesh,
        compiler_params=pltpu.CompilerParams(needs_layout_passes=False),
        scratch_types=[
            pltpu.VMEM((_RPW * _N,), jnp.float32),
            pltpu.VMEM((_N,), jnp.uint32),
            pltpu.VMEM((_NBINS,), jnp.int32),
            pltpu.VMEM((_N,), jnp.int32),
            pltpu.VMEM((_N,), jnp.int32),
        ],
    )
    return f(input)


# speedup vs baseline: 2.1940x; 1.1149x over previous
import jax
import jax.numpy as jnp
from jax import lax
from jax.experimental import pallas as pl
from jax.experimental.pallas import tpu as pltpu
from jax.experimental.pallas import tpu_sc as plsc

_B, _N = 64, 8192
_K = _N // 2
_NC, _NS = 2, 16
_NW = _NC * _NS
_B_SC = 32
_B_TC = _B - _B_SC
_RPW = _B_SC // _NW
_L = 16
_VPR = _N // _L
_NBINS = 256
_BIG = 2 ** 30


def _scan_select(hist_ref, kk):
    carry = jnp.int32(0)
    cnt_vec = jnp.zeros((_L,), jnp.int32)
    a_vec = jnp.zeros((_L,), jnp.int32)
    m_vec = jnp.full((_L,), _BIG, jnp.int32)
    for j in range(_NBINS // _L):
        v = hist_ref[pl.ds((_NBINS // _L - 1 - j) * _L, _L)]
        rv = lax.rev(v, (0,))
        cs = plsc.cumsum(rv) + carry
        ge = cs >= kk
        cnt_vec = cnt_vec + ge.astype(jnp.int32)
        a_vec = jnp.maximum(a_vec, jnp.where(ge, 0, cs))
        m_vec = jnp.minimum(m_vec, jnp.where(ge, cs, _BIG))
        carry = jnp.max(cs)
    b = jnp.sum(cnt_vec) - 1
    return b, jnp.max(a_vec), jnp.min(m_vec)


def _zero_hist(hist_ref):
    zeros = jnp.zeros((_L,), jnp.int32)
    for i in range(_NBINS // _L):
        hist_ref[pl.ds(i * _L, _L)] = zeros


def _process_row(xrow, krow, hist_ref, cb1, cb2):
    ones = jnp.ones((_L,), jnp.int32)

    _zero_hist(hist_ref)

    @plsc.parallel_loop(0, _N, step=_L, unroll=8)
    def body1(i):
        xvec = xrow[pl.ds(i, _L)]
        u = lax.bitcast_convert_type(xvec, jnp.uint32)
        ku = jnp.where(u >= jnp.uint32(0x80000000), ~u, u | jnp.uint32(0x80000000))
        krow[pl.ds(i, _L)] = ku
        digit = jnp.right_shift(ku, jnp.uint32(24)).astype(jnp.int32)
        plsc.addupdate_scatter(hist_ref, [digit], ones)

    kk = jnp.int32(_K)
    b, A, M = _scan_select(hist_ref, kk)
    kk = kk - A
    prefix = b.astype(jnp.uint32)

    _zero_hist(hist_ref)

    @plsc.parallel_loop(0, _N, step=_L, unroll=8,
                        carry=jnp.zeros((_L,), jnp.int32))
    def body2(i, off_vec):
        kvec = krow[pl.ds(i, _L)]
        cand = jnp.right_shift(kvec, jnp.uint32(24)) == prefix
        digit = (jnp.right_shift(kvec, jnp.uint32(16))
                 & jnp.uint32(0xFF)).astype(jnp.int32)
        plsc.addupdate_scatter(hist_ref, [digit], ones, mask=cand)
        rank = plsc.cumsum(cand.astype(jnp.int32))
        idx = off_vec + rank - 1
        plsc.store_scatter(cb1, [idx], lax.bitcast_convert_type(kvec, jnp.int32),
                           mask=cand)
        return off_vec + plsc.all_reduce_population_count(cand)

    ncand1 = jnp.max(body2)
    b, A, M = _scan_select(hist_ref, kk)
    kk = kk - A
    prefix = jnp.left_shift(prefix, jnp.uint32(8)) | b.astype(jnp.uint32)

    iota = lax.broadcasted_iota(jnp.int32, (_L,), 0)
    _zero_hist(hist_ref)

    def body3(i, off_vec):
        base = i * _L
        kvec = lax.bitcast_convert_type(cb1[pl.ds(base, _L)], jnp.uint32)
        valid = (iota + base) < ncand1
        cand = valid & (jnp.right_shift(kvec, jnp.uint32(16)) == prefix)
        digit = (jnp.right_shift(kvec, jnp.uint32(8))
                 & jnp.uint32(0xFF)).astype(jnp.int32)
        plsc.addupdate_scatter(hist_ref, [digit], ones, mask=cand)
        rank = plsc.cumsum(cand.astype(jnp.int32))
        idx = off_vec + rank - 1
        plsc.store_scatter(cb2, [idx], lax.bitcast_convert_type(kvec, jnp.int32),
                           mask=cand)
        return off_vec + plsc.all_reduce_population_count(cand)

    nvec3 = (ncand1 + (_L - 1)) // _L
    ncand2 = jnp.max(lax.fori_loop(0, nvec3, body3,
                                   jnp.zeros((_L,), jnp.int32)))
    b, A, M = _scan_select(hist_ref, kk)
    kk = kk - A
    prefix = jnp.left_shift(prefix, jnp.uint32(8)) | b.astype(jnp.uint32)

    _zero_hist(hist_ref)

    def body4(i, c):
        base = i * _L
        kvec = lax.bitcast_convert_type(cb2[pl.ds(base, _L)], jnp.uint32)
        cand = (((iota + base) < ncand2)
                & (jnp.right_shift(kvec, jnp.uint32(8)) == prefix))
        digit = (kvec & jnp.uint32(0xFF)).astype(jnp.int32)
        plsc.addupdate_scatter(hist_ref, [digit], ones, mask=cand)
        return c

    nvec4 = (ncand2 + (_L - 1)) // _L
    lax.fori_loop(0, nvec4, body4, 0)
    b, A, M = _scan_select(hist_ref, kk)
    kk = kk - A
    prefix = jnp.left_shift(prefix, jnp.uint32(8)) | b.astype(jnp.uint32)

    t = prefix
    rem = (M - A) - kk

    @plsc.parallel_loop(0, _N, step=_L, unroll=8)
    def body_mask(i):
        kvec = krow[pl.ds(i, _L)]
        xvec = xrow[pl.ds(i, _L)]
        xrow[pl.ds(i, _L)] = jnp.where(kvec >= t, xvec, jnp.float32(0.0))

    @pl.when(rem > 0)
    def _fixup():
        def bodyf(j, r):
            i = _VPR - 1 - j
            kvec = krow[pl.ds(i * _L, _L)]
            eq = kvec == t
            rcs = plsc.cumsum(lax.rev(eq.astype(jnp.int32), (0,)))
            zmask = eq & (lax.rev(rcs, (0,)) <= r)
            xvec = xrow[pl.ds(i * _L, _L)]
            xrow[pl.ds(i * _L, _L)] = jnp.where(zmask, jnp.float32(0.0), xvec)
            return r - jnp.max(rcs)

        lax.fori_loop(0, _VPR, bodyf, rem)


def _sc_body(x_hbm, o_hbm, xv, kv, hist, cb1, cb2):
    wid = lax.axis_index("s") * _NC + lax.axis_index("c")
    base = wid * _RPW
    for r in range(_RPW):
        pltpu.sync_copy(x_hbm.at[base + r], xv.at[pl.ds(r * _N, _N)])
    for r in range(_RPW):
        _process_row(xv.at[pl.ds(r * _N, _N)], kv, hist, cb1, cb2)
    for r in range(_RPW):
        pltpu.sync_copy(xv.at[pl.ds(r * _N, _N)], o_hbm.at[base + r])


def _tc_topk_mask(x_ref, o_ref):
    x = x_ref[...]
    u = lax.bitcast_convert_type(x, jnp.uint32)
    sign = jnp.uint32(0x80000000)
    ku = jnp.where(u >= sign, ~u, u | sign)

    def body(s, prefix):
        bit = jnp.left_shift(jnp.uint32(1), jnp.uint32(31) - s.astype(jnp.uint32))
        tryv = prefix | bit
        c = jnp.sum((ku >= tryv).astype(jnp.int32), axis=1, keepdims=True)
        return jnp.where(c >= _K, tryv, prefix)

    T = lax.fori_loop(0, 32, body, jnp.zeros((x.shape[0], 1), jnp.uint32))

    gt = ku > T
    eq = ku == T
    count_gt = jnp.sum(gt.astype(jnp.int32), axis=1, keepdims=True)
    need = _K - count_gt

    csum = eq.astype(jnp.int32)
    d = 1
    while d < _N:
        shifted = jnp.concatenate(
            [jnp.zeros((x.shape[0], d), jnp.int32), csum[:, : _N - d]], axis=1
        )
        csum = csum + shifted
        d *= 2

    mask = gt | (eq & (csum <= need))
    o_ref[...] = x * mask.astype(x.dtype)


@jax.jit
def kernel(input):
    mesh = plsc.VectorSubcoreMesh(
        core_axis_name="c", subcore_axis_name="s",
        num_cores=_NC, num_subcores=_NS,
    )
    f = pl.kernel(
        _sc_body,
        out_type=jax.ShapeDtypeStruct((_B_SC, _N), jnp.float32),
        mesh=mesh,
        compiler_params=pltpu.CompilerParams(needs_layout_passes=False),
        scratch_types=[
            pltpu.VMEM((_RPW * _N,), jnp.float32),
            pltpu.VMEM((_N,), jnp.uint32),
            pltpu.VMEM((_NBINS,), jnp.int32),
            pltpu.VMEM((_N,), jnp.int32),
            pltpu.VMEM((_N,), jnp.int32),
        ],
    )
    out_sc = f(input[_B_TC:])
    out_tc = pl.pallas_call(
        _tc_topk_mask,
        out_shape=jax.ShapeDtypeStruct((_B_TC, _N), jnp.float32),
    )(input[:_B_TC])
    return jnp.concatenate([out_tc, out_sc], axis=0)


# speedup vs baseline: 3.8658x; 1.7620x over previous
import functools

import jax
import jax.numpy as jnp
from jax.experimental import pallas as pl
from jax.experimental.pallas import tpu as pltpu


def _topk_mask_kernel(x_ref, o_ref, *, k):
    x = x_ref[...]
    n = x.shape[-1]
    u = jax.lax.bitcast_convert_type(x, jnp.uint32)
    sign = jnp.uint32(0x80000000)
    ku = jnp.where(u >= sign, ~u, u | sign)

    def body(t, prefix):
        bit = jnp.left_shift(jnp.uint32(1), jnp.uint32(31) - t.astype(jnp.uint32))
        tryv = prefix | bit
        c = jnp.sum((ku >= tryv).astype(jnp.int32), axis=1, keepdims=True)
        return jnp.where(c >= k, tryv, prefix)

    T = jax.lax.fori_loop(0, 32, body, jnp.zeros((x.shape[0], 1), jnp.uint32))

    gt = ku > T
    eq = ku == T
    count_gt = jnp.sum(gt.astype(jnp.int32), axis=1, keepdims=True)
    need = k - count_gt

    csum = eq.astype(jnp.int32)
    d = 1
    while d < n:
        shifted = jnp.concatenate(
            [jnp.zeros((x.shape[0], d), jnp.int32), csum[:, : n - d]], axis=1
        )
        csum = csum + shifted
        d *= 2

    mask = gt | (eq & (csum <= need))
    o_ref[...] = x * mask.astype(x.dtype)


@jax.jit
def kernel(input):
    b, n = input.shape
    k = n // 2
    return pl.pallas_call(
        functools.partial(_topk_mask_kernel, k=k),
        out_shape=jax.ShapeDtypeStruct((b, n), input.dtype),
    )(input)
